# baseline (device time: 6988 ns/iter reference)
import jax
import jax.numpy as jnp
from jax import lax
from jax.experimental import pallas as pl
from jax.experimental.pallas import tpu as pltpu

N_GLOBAL = 1024


def kernel(x):
    m_per, n_per = x.shape
    rows, cols = 8, m_per // 8

    def body(x_ref, out_ref, comm_ref, send_sem, recv_sem):
        my_x = lax.axis_index("x")
        my_y = lax.axis_index("y")
        peer = (my_x, 1 - my_y)

        barrier_sem = pltpu.get_barrier_semaphore()
        pl.semaphore_signal(
            barrier_sem, inc=1,
            device_id=peer, device_id_type=pl.DeviceIdType.MESH,
        )
        pl.semaphore_wait(barrier_sem, 1)

        xb = x_ref[:, :].astype(jnp.bfloat16)
        ones = jnp.ones((n_per, cols), jnp.bfloat16)
        s = jax.lax.dot_general(
            xb, ones, (((1,), (0,)), ((), ())),
            preferred_element_type=jnp.float32,
        )

        rl = lax.broadcasted_iota(jnp.int32, (m_per, cols), 0)
        jl = lax.broadcasted_iota(jnp.int32, (m_per, cols), 1)
        sm = jnp.where(rl % cols == jl, s, 0.0)

        kt = lax.broadcasted_iota(jnp.int32, (rows, m_per), 0)
        rt = lax.broadcasted_iota(jnp.int32, (rows, m_per), 1)
        e_t = (rt // cols == kt).astype(jnp.float32)
        comm_ref[0, :, :] = jax.lax.dot_general(
            e_t, sm, (((1,), (0,)), ((), ())),
            preferred_element_type=jnp.float32,
        )

        rdma = pltpu.make_async_remote_copy(
            src_ref=comm_ref.at[0],
            dst_ref=comm_ref.at[1],
            send_sem=send_sem,
            recv_sem=recv_sem,
            device_id=peer,
            device_id_type=pl.DeviceIdType.MESH,
        )
        rdma.start()

        local_out = jnp.sum(sm, axis=1, keepdims=True)
        rk = lax.broadcasted_iota(jnp.int32, (m_per, rows), 0)
        kk = lax.broadcasted_iota(jnp.int32, (m_per, rows), 1)
        e = (rk // cols == kk).astype(jnp.float32)

        rdma.wait_recv()

        u = jax.lax.dot_general(
            e, comm_ref[1, :, :], (((1,), (0,)), ((), ())),
            preferred_element_type=jnp.float32,
        )
        peer_out = jnp.sum(
            jnp.where(rl % cols == jl, u, 0.0), axis=1, keepdims=True
        )
        out_ref[:, :] = (local_out + peer_out) * (1.0 / N_GLOBAL)

        rdma.wait_send()

    return pl.pallas_call(
        body,
        out_shape=jax.ShapeDtypeStruct((m_per, 1), jnp.float32),
        in_specs=[pl.BlockSpec(memory_space=pltpu.VMEM)],
        out_specs=pl.BlockSpec(memory_space=pltpu.VMEM),
        scratch_shapes=[
            pltpu.VMEM((2, rows, cols), jnp.float32),
            pltpu.SemaphoreType.DMA,
            pltpu.SemaphoreType.DMA,
        ],
        compiler_params=pltpu.CompilerParams(collective_id=0),
    )(x)


# device time: 2434 ns/iter; 2.8710x vs baseline; 2.8710x over previous
import jax
import jax.numpy as jnp
from jax import lax
from jax.experimental import pallas as pl
from jax.experimental.pallas import tpu as pltpu

N_GLOBAL = 1024


def kernel(x):
    m_per, n_per = x.shape
    rows, cols = 8, m_per // 8

    def body(x_ref, out_ref, comm_ref, send_sem, recv_sem):
        my_x = lax.axis_index("x")
        my_y = lax.axis_index("y")
        peer = (my_x, 1 - my_y)

        barrier_sem = pltpu.get_barrier_semaphore()
        pl.semaphore_signal(
            barrier_sem, inc=1,
            device_id=peer, device_id_type=pl.DeviceIdType.MESH,
        )
        pl.semaphore_wait(barrier_sem, 1)

        x3 = x_ref[:, :].reshape(rows, cols, n_per)
        comm_ref[0, :, :] = jnp.sum(x3, axis=2)

        rdma = pltpu.make_async_remote_copy(
            src_ref=comm_ref.at[0],
            dst_ref=comm_ref.at[1],
            send_sem=send_sem,
            recv_sem=recv_sem,
            device_id=peer,
            device_id_type=pl.DeviceIdType.MESH,
        )
        rdma.start()

        r8 = lax.broadcasted_iota(jnp.int32, (m_per, rows), 0)
        k8 = lax.broadcasted_iota(jnp.int32, (m_per, rows), 1)
        sel = (r8 // cols == k8).astype(jnp.float32)
        rl = lax.broadcasted_iota(jnp.int32, (m_per, cols), 0)
        jl = lax.broadcasted_iota(jnp.int32, (m_per, cols), 1)
        lane_mask = (rl % cols == jl).astype(jnp.float32)

        spread_mine = jax.lax.dot_general(
            sel, comm_ref[0, :, :], (((1,), (0,)), ((), ())),
            preferred_element_type=jnp.float32,
        )
        out_mine = jnp.sum(spread_mine * lane_mask, axis=1, keepdims=True)

        rdma.wait_recv()

        spread_peer = jax.lax.dot_general(
            sel, comm_ref[1, :, :], (((1,), (0,)), ((), ())),
            preferred_element_type=jnp.float32,
        )
        out_peer = jnp.sum(spread_peer * lane_mask, axis=1, keepdims=True)
        out_ref[:, :] = (out_mine + out_peer) * (1.0 / N_GLOBAL)

        rdma.wait_send()

    return pl.pallas_call(
        body,
        out_shape=jax.ShapeDtypeStruct((m_per, 1), jnp.float32),
        in_specs=[pl.BlockSpec(memory_space=pltpu.VMEM)],
        out_specs=pl.BlockSpec(memory_space=pltpu.VMEM),
        scratch_shapes=[
            pltpu.VMEM((2, rows, cols), jnp.float32),
            pltpu.SemaphoreType.DMA,
            pltpu.SemaphoreType.DMA,
        ],
        compiler_params=pltpu.CompilerParams(collective_id=0),
    )(x)
